# Initial kernel scaffold; baseline (speedup 1.0000x reference)
import jax
import jax.numpy as jnp
from jax.experimental import pallas as pl


def kernel(words_idx, num_valid_words, bigram, trigram, emb, emb2, emb3, W1, b1, W2, b2):
    raise NotImplementedError("write your pallas kernel here")


# speedup vs baseline: 2.2453x; 2.2453x over previous
import numpy as np
import jax
import jax.numpy as jnp
from jax import lax
from jax.experimental import pallas as pl
from jax.experimental.pallas import tpu as pltpu
from jax.experimental.pallas import tpu_sc as plsc

NC = 2
NS = 16
NW = NC * NS
BATCH = 4096
PAD = 20
EMBED = 128
HIDDEN = 1024
LABELS = 100
SPW = BATCH // NW
ROWS = SPW * PAD
CHUNK = 128
NCH = ROWS // CHUNK
ACC = NS * SPW


def _dest_idx_table():
    q = (np.arange(ROWS, dtype=np.int32) // PAD).reshape(NCH, CHUNK)
    out = np.empty((NS, 3, NCH, CHUNK), dtype=np.int32)
    for s in range(NS):
        for t in range(3):
            out[s, t] = t * ACC + s * SPW + q
    return out.reshape(NS, 3 * NCH, CHUNK)


_DEST = _dest_idx_table()


def _pool_sc(emb, emb2, emb3, gi1, gi2, gi3, dest, zeros):
    mesh = plsc.VectorSubcoreMesh(core_axis_name="c", subcore_axis_name="s")
    out_type = tuple(
        jax.ShapeDtypeStruct((BATCH, EMBED), jnp.float32) for _ in range(3)
    )

    def body(emb_h, emb2_h, emb3_h, gi1_h, gi2_h, gi3_h, dest_h, zeros_h,
             m1_h, m2_h, m3_h, gidx_v, didx_v, buf, acc_sh, gsem):
        c = lax.axis_index("c")
        s = lax.axis_index("s")
        wid = s * NC + c

        pltpu.sync_copy(gi1_h.at[wid], gidx_v.at[0])
        pltpu.sync_copy(gi2_h.at[wid], gidx_v.at[1])
        pltpu.sync_copy(gi3_h.at[wid], gidx_v.at[2])
        pltpu.sync_copy(dest_h.at[s], didx_v)

        for t in range(3):
            pltpu.sync_copy(zeros_h, acc_sh.at[pl.ds(t * ACC + s * SPW, SPW)])

        tables = (emb_h, emb2_h, emb3_h)
        for t in range(3):
            for j in range(NCH):
                b = j % 2
                pltpu.async_copy(
                    tables[t].at[gidx_v.at[t, j]], buf.at[b], gsem.at[b]
                ).wait()
                pltpu.sync_copy(
                    buf.at[b], acc_sh.at[didx_v.at[t * NCH + j]], add=True
                )

        for t, m_h in enumerate((m1_h, m2_h, m3_h)):
            pltpu.sync_copy(
                acc_sh.at[pl.ds(t * ACC + s * SPW, SPW)],
                m_h.at[pl.ds(wid * SPW, SPW)],
            )

    return pl.kernel(
        body,
        out_type=out_type,
        mesh=mesh,
        scratch_types=[
            pltpu.VMEM((3, NCH, CHUNK), jnp.int32),
            pltpu.VMEM((3 * NCH, CHUNK), jnp.int32),
            pltpu.VMEM((2, CHUNK, EMBED), jnp.float32),
            pltpu.VMEM_SHARED((3 * ACC, EMBED), jnp.float32),
            pltpu.SemaphoreType.DMA((2,)),
        ],
    )(emb, emb2, emb3, gi1, gi2, gi3, dest, zeros)


def _mlp_body(m1_r, m2_r, m3_r, a1_r, a2_r, a3_r, b1_r, w2_r, b2_r, o_r):
    h = jnp.dot(m1_r[...], a1_r[...], preferred_element_type=jnp.float32)
    h += jnp.dot(m2_r[...], a2_r[...], preferred_element_type=jnp.float32)
    h += jnp.dot(m3_r[...], a3_r[...], preferred_element_type=jnp.float32)
    h = jnp.maximum(h * (1.0 / PAD) + b1_r[...], 0.0)
    o_r[...] = jnp.dot(h, w2_r[...], preferred_element_type=jnp.float32) + b2_r[...]


def _mlp_tc(m1, m2, m3, W1, b1, W2, b2):
    a1 = W1[:EMBED]
    a2 = W1[EMBED:2 * EMBED]
    a3 = W1[2 * EMBED:]
    w2p = jnp.zeros((HIDDEN, 128), W2.dtype).at[:, :LABELS].set(W2)
    b2p = jnp.zeros((1, 128), b2.dtype).at[0, :LABELS].set(b2)
    b1r = b1.reshape(1, HIDDEN)
    tile = 512
    grid = BATCH // tile
    full = lambda shape: pl.BlockSpec(shape, lambda i: (0, 0))
    out = pl.pallas_call(
        _mlp_body,
        grid=(grid,),
        in_specs=[
            pl.BlockSpec((tile, EMBED), lambda i: (i, 0)),
            pl.BlockSpec((tile, EMBED), lambda i: (i, 0)),
            pl.BlockSpec((tile, EMBED), lambda i: (i, 0)),
            full((EMBED, HIDDEN)),
            full((EMBED, HIDDEN)),
            full((EMBED, HIDDEN)),
            full((1, HIDDEN)),
            full((HIDDEN, 128)),
            full((1, 128)),
        ],
        out_specs=pl.BlockSpec((tile, 128), lambda i: (i, 0)),
        out_shape=jax.ShapeDtypeStruct((BATCH, 128), jnp.float32),
    )(m1, m2, m3, a1, a2, a3, b1r, w2p, b2p)
    return out[:, :LABELS]


def kernel(words_idx, num_valid_words, bigram, trigram, emb, emb2, emb3,
           W1, b1, W2, b2):
    gi1 = words_idx.astype(jnp.int32).reshape(NW, NCH, CHUNK)
    gi2 = bigram.astype(jnp.int32).reshape(NW, NCH, CHUNK)
    gi3 = trigram.astype(jnp.int32).reshape(NW, NCH, CHUNK)
    dest = jnp.asarray(_DEST)
    zeros = jnp.zeros((SPW, EMBED), jnp.float32)
    m1, m2, m3 = _pool_sc(emb, emb2, emb3, gi1, gi2, gi3, dest, zeros)
    return _mlp_tc(m1, m2, m3, W1, b1, W2, b2)
